# baseline (speedup 1.0000x reference)
import functools

import jax
import jax.numpy as jnp
from jax import lax
from jax.experimental import pallas as pl
from jax.experimental.pallas import tpu as pltpu
from jax.experimental.pallas import tpu_sc as plsc

_B, _L, _K, _D, _OUT = 1024, 50, 5, 128, 8
_PAD = 1
_NC, _NS = 2, 16
_NW = _NC * _NS
_NIDX = _B * _L
_CHUNK = 100
_NCHUNK = _NIDX // (_NW * _CHUNK)

_BB = 64
_NEG = -1e18


def _ir_gather_sc(table, idx):
    mesh = plsc.VectorSubcoreMesh(core_axis_name="c", subcore_axis_name="s")

    @functools.partial(
        pl.kernel,
        mesh=mesh,
        out_type=jax.ShapeDtypeStruct((_NW, _NCHUNK, _CHUNK), jnp.float32),
        scratch_types=[
            pltpu.VMEM((_NCHUNK, _CHUNK), jnp.int32),
            pltpu.VMEM((_NCHUNK, _CHUNK), jnp.float32),
            pltpu.SemaphoreType.DMA,
        ],
    )
    def gather_kernel(table_hbm, idx_hbm, out_hbm, idx_v, vals_v, sem):
        wid = lax.axis_index("s") * _NC + lax.axis_index("c")
        pltpu.sync_copy(idx_hbm.at[wid], idx_v)
        copies = [
            pltpu.async_copy(table_hbm.at[idx_v.at[j]], vals_v.at[j], sem)
            for j in range(_NCHUNK)
        ]
        for c in copies:
            c.wait()
        pltpu.sync_copy(vals_v, out_hbm.at[wid])

    return gather_kernel(table, idx)


def _tc_body(ns_ref, x_ref, ew_ref, nbr_ref, ir_ref, w_ref, b_ref, out_ref):
    m = None
    for k in range(_K):
        t = ew_ref[k] * nbr_ref[:, :, k * _D:(k + 1) * _D]
        t = jnp.where(t == 0.0, _NEG, t)
        m = t if m is None else jnp.maximum(m, t)
    ir = jnp.where(ns_ref[...] == _PAD, 1.0, ir_ref[...])
    emb = (1.0 - ir) * m + ir * x_ref[...]
    s = jnp.sum(emb, axis=1)
    z = lax.dot_general(s, w_ref[...], (((1,), (1,)), ((), ())),
                        preferred_element_type=jnp.float32)
    z = jnp.maximum(z + b_ref[...], 0.0)
    z = z - jnp.max(z, axis=1, keepdims=True)
    e = jnp.exp(z)
    out_ref[...] = e / jnp.sum(e, axis=1, keepdims=True)


def _tc_call(ns3, x, ew_t, nbr2, ir3, W, b2):
    return pl.pallas_call(
        _tc_body,
        grid=(_B // _BB,),
        in_specs=[
            pl.BlockSpec((_BB, _L, 1), lambda i: (i, 0, 0)),
            pl.BlockSpec((_BB, _L, _D), lambda i: (i, 0, 0)),
            pl.BlockSpec((_K, _BB, _L, 1), lambda i: (0, i, 0, 0)),
            pl.BlockSpec((_BB, _L, _K * _D), lambda i: (i, 0, 0)),
            pl.BlockSpec((_BB, _L, 1), lambda i: (i, 0, 0)),
            pl.BlockSpec((_OUT, _D), lambda i: (0, 0)),
            pl.BlockSpec((1, _OUT), lambda i: (0, 0)),
        ],
        out_specs=pl.BlockSpec((_BB, _OUT), lambda i: (i, 0)),
        out_shape=jax.ShapeDtypeStruct((_B, _OUT), jnp.float32),
    )(ns3, x, ew_t, nbr2, ir3, W, b2)


def kernel(node_sets, embedded_node, edge_weight, embedded_neighbor_node,
           information_rate, W, b):
    ns = jnp.asarray(node_sets, jnp.int32)
    table = information_rate.reshape(-1)
    ir = _ir_gather_sc(table, ns.reshape(_NW, _NCHUNK, _CHUNK))
    ir3 = ir.reshape(_B, _L, 1)
    ns3 = ns.reshape(_B, _L, 1)
    ew_t = jnp.transpose(edge_weight, (2, 0, 1))[..., None]
    nbr2 = embedded_neighbor_node.reshape(_B, _L, _K * _D)
    b2 = b.reshape(1, _OUT)
    return _tc_call(ns3, embedded_node, ew_t, nbr2, ir3, W, b2)


# speedup vs baseline: 1.5701x; 1.5701x over previous
import functools

import jax
import jax.numpy as jnp
from jax import lax
from jax.experimental import pallas as pl
from jax.experimental.pallas import tpu as pltpu
from jax.experimental.pallas import tpu_sc as plsc

_B, _L, _K, _D, _OUT = 1024, 50, 5, 128, 8
_PAD = 1
_NC, _NS = 2, 16
_NW = _NC * _NS
_NIDX = _B * _L
_CHUNK = 100
_NCHUNK = _NIDX // (_NW * _CHUNK)

_BB = 64
_NEG = -1e18


def _ir_gather_sc(table, idx):
    mesh = plsc.VectorSubcoreMesh(core_axis_name="c", subcore_axis_name="s")

    @functools.partial(
        pl.kernel,
        mesh=mesh,
        out_type=jax.ShapeDtypeStruct((_NW, _NCHUNK, _CHUNK), jnp.float32),
        scratch_types=[
            pltpu.VMEM((_NCHUNK, _CHUNK), jnp.int32),
            pltpu.VMEM((_NCHUNK, _CHUNK), jnp.float32),
            pltpu.SemaphoreType.DMA,
        ],
    )
    def gather_kernel(table_hbm, idx_hbm, out_hbm, idx_v, vals_v, sem):
        wid = lax.axis_index("s") * _NC + lax.axis_index("c")
        pltpu.sync_copy(idx_hbm.at[wid], idx_v)
        copies = [
            pltpu.async_copy(table_hbm.at[idx_v.at[j]], vals_v.at[j], sem)
            for j in range(_NCHUNK)
        ]
        for c in copies:
            c.wait()
        pltpu.sync_copy(vals_v, out_hbm.at[wid])

    return gather_kernel(table, idx)


def _tc_body(ns_ref, x_ref, ew_ref, nbr_ref, ir_ref, w_ref, b_ref, out_ref):
    tmp = ew_ref[...][..., None] * nbr_ref[...]
    tmp = jnp.where(tmp == 0.0, _NEG, tmp)
    m = jnp.max(tmp, axis=2)
    g = jnp.where(ns_ref[...] == _PAD, 1.0, ir_ref[...])
    gb = g[:, :, None]
    emb = (1.0 - gb) * m + gb * x_ref[...]
    s = jnp.sum(emb, axis=1)
    z = lax.dot_general(s, w_ref[...], (((1,), (1,)), ((), ())),
                        preferred_element_type=jnp.float32)
    z = jnp.maximum(z + b_ref[...], 0.0)
    z = z - jnp.max(z, axis=1, keepdims=True)
    e = jnp.exp(z)
    out_ref[...] = e / jnp.sum(e, axis=1, keepdims=True)


def _tc_call(ns2, x, ew, nbr, ir2, W, b2):
    return pl.pallas_call(
        _tc_body,
        grid=(_B // _BB,),
        in_specs=[
            pl.BlockSpec((_BB, _L), lambda i: (i, 0)),
            pl.BlockSpec((_BB, _L, _D), lambda i: (i, 0, 0)),
            pl.BlockSpec((_BB, _L, _K), lambda i: (i, 0, 0)),
            pl.BlockSpec((_BB, _L, _K, _D), lambda i: (i, 0, 0, 0)),
            pl.BlockSpec((_BB, _L), lambda i: (i, 0)),
            pl.BlockSpec((_OUT, _D), lambda i: (0, 0)),
            pl.BlockSpec((1, _OUT), lambda i: (0, 0)),
        ],
        out_specs=pl.BlockSpec((_BB, _OUT), lambda i: (i, 0)),
        out_shape=jax.ShapeDtypeStruct((_B, _OUT), jnp.float32),
    )(ns2, x, ew, nbr, ir2, W, b2)


def kernel(node_sets, embedded_node, edge_weight, embedded_neighbor_node,
           information_rate, W, b):
    ns = jnp.asarray(node_sets, jnp.int32)
    table = information_rate.reshape(-1)
    ir = _ir_gather_sc(table, ns.reshape(_NW, _NCHUNK, _CHUNK))
    ir2 = ir.reshape(_B, _L)
    b2 = b.reshape(1, _OUT)
    return _tc_call(ns, embedded_node, edge_weight, embedded_neighbor_node,
                    ir2, W, b2)


# speedup vs baseline: 5.2856x; 3.3664x over previous
import functools

import jax
import jax.numpy as jnp
from jax import lax
from jax.experimental import pallas as pl
from jax.experimental.pallas import tpu as pltpu
from jax.experimental.pallas import tpu_sc as plsc

_B, _L, _K, _D, _OUT = 1024, 50, 5, 128, 8
_PAD = 1
_NC, _NS = 2, 16
_NW = _NC * _NS
_NIDX = _B * _L
_CHUNK = 100
_NCHUNK = _NIDX // (_NW * _CHUNK)

_BB = 128
_NEG = -1e18


def _ir_gather_sc(table, idx):
    mesh = plsc.VectorSubcoreMesh(core_axis_name="c", subcore_axis_name="s")

    @functools.partial(
        pl.kernel,
        mesh=mesh,
        out_type=jax.ShapeDtypeStruct((_NW, _NCHUNK, _CHUNK), jnp.float32),
        scratch_types=[
            pltpu.VMEM((_NCHUNK, _CHUNK), jnp.int32),
            pltpu.VMEM((_NCHUNK, _CHUNK), jnp.float32),
            pltpu.SemaphoreType.DMA,
        ],
    )
    def gather_kernel(table_hbm, idx_hbm, out_hbm, idx_v, vals_v, sem):
        wid = lax.axis_index("s") * _NC + lax.axis_index("c")
        pltpu.sync_copy(idx_hbm.at[wid], idx_v)
        copies = [
            pltpu.async_copy(table_hbm.at[idx_v.at[j]], vals_v.at[j], sem)
            for j in range(_NCHUNK)
        ]
        for c in copies:
            c.wait()
        pltpu.sync_copy(vals_v, out_hbm.at[wid])

    return gather_kernel(table, idx)


def _tc_body(ns_ref, x_ref, ew_ref, nbr_ref, ir_ref, w_ref, b_ref, out_ref):
    m = None
    for k in range(_K):
        t = ew_ref[k][:, :, None] * nbr_ref[:, k]
        t = jnp.where(t == 0.0, _NEG, t)
        m = t if m is None else jnp.maximum(m, t)
    g = jnp.where(ns_ref[...] == _PAD, 1.0, ir_ref[...])
    gb = g[:, :, None]
    emb = (1.0 - gb) * m + gb * x_ref[...]
    s = jnp.sum(emb, axis=0)
    z = lax.dot_general(s, w_ref[...], (((1,), (1,)), ((), ())),
                        preferred_element_type=jnp.float32)
    z = jnp.maximum(z + b_ref[...], 0.0)
    z = z - jnp.max(z, axis=1, keepdims=True)
    e = jnp.exp(z)
    out_ref[...] = e / jnp.sum(e, axis=1, keepdims=True)


def _tc_call(ns_t, x_t, ew_t, nbr_t, ir_t, W, b2):
    return pl.pallas_call(
        _tc_body,
        grid=(_B // _BB,),
        in_specs=[
            pl.BlockSpec((_L, _BB), lambda i: (0, i)),
            pl.BlockSpec((_L, _BB, _D), lambda i: (0, i, 0)),
            pl.BlockSpec((_K, _L, _BB), lambda i: (0, 0, i)),
            pl.BlockSpec((_L, _K, _BB, _D), lambda i: (0, 0, i, 0)),
            pl.BlockSpec((_L, _BB), lambda i: (0, i)),
            pl.BlockSpec((_OUT, _D), lambda i: (0, 0)),
            pl.BlockSpec((1, _OUT), lambda i: (0, 0)),
        ],
        out_specs=pl.BlockSpec((_BB, _OUT), lambda i: (i, 0)),
        out_shape=jax.ShapeDtypeStruct((_B, _OUT), jnp.float32),
    )(ns_t, x_t, ew_t, nbr_t, ir_t, W, b2)


def kernel(node_sets, embedded_node, edge_weight, embedded_neighbor_node,
           information_rate, W, b):
    ns_t = jnp.transpose(jnp.asarray(node_sets, jnp.int32), (1, 0))
    x_t = jnp.transpose(embedded_node, (1, 0, 2))
    ew_t = jnp.transpose(edge_weight, (2, 1, 0))
    nbr_t = jnp.transpose(embedded_neighbor_node, (1, 2, 0, 3))
    table = information_rate.reshape(-1)
    ir_t = _ir_gather_sc(table, ns_t.reshape(_NW, _NCHUNK, _CHUNK)).reshape(_L, _B)
    b2 = b.reshape(1, _OUT)
    return _tc_call(ns_t, x_t, ew_t, nbr_t, ir_t, W, b2)
